# baseline (speedup 1.0000x reference)
import functools

import jax
import jax.numpy as jnp
from jax import lax
from jax.experimental import pallas as pl
from jax.experimental.pallas import tpu as pltpu
from jax.experimental.pallas import tpu_sc as plsc

_B = 4096
_V = 1000
_NC = 2
_NS = 16
_NW = _NC * _NS
_RPW = _B // _NW
_CH = 32
_NCH = _RPW // _CH
_NVR = _V // 16
_TAIL = _V - 16


def _sc_body(out_hbm, labels_hbm, m_hbm, s_hbm, g_hbm,
             xbuf0, xbuf1, lbuf, mbuf, sbuf, gbuf, gsem):
    xbufs = (xbuf0, xbuf1)
    c = lax.axis_index("c")
    s_ax = lax.axis_index("s")
    wid = s_ax * _NC + c
    base = wid * _RPW
    iota = lax.iota(jnp.int32, 16)

    gd = lax.GatherDimensionNumbers(
        offset_dims=(), collapsed_slice_dims=(0,), start_index_map=(0,))

    def xlane(v, idx):
        return lax.gather(v, idx[:, None], gd, slice_sizes=(1,),
                          mode=lax.GatherScatterMode.PROMISE_IN_BOUNDS)

    def lane_max(v):
        for k in (8, 4, 2, 1):
            v = jnp.maximum(v, xlane(v, jnp.bitwise_xor(iota, k)))
        return v

    def lane_sum(v):
        for k in (8, 4, 2, 1):
            v = v + xlane(v, jnp.bitwise_xor(iota, k))
        return v

    def start_gather(ch, slot):
        return [
            pltpu.async_copy(
                out_hbm.at[pl.ds((base + ch * _CH + rl) * 8 * _V + 7 * _V,
                               _V)],
                xbufs[slot].at[pl.ds(rl * _V, _V)],
                gsem.at[slot],
            )
            for rl in range(_CH)
        ]

    cps = [start_gather(0, 0)]
    pltpu.sync_copy(labels_hbm.at[pl.ds(base * 16, _RPW * 16)],
                    lbuf.at[pl.ds(0, _RPW * 16)])

    for ch in range(_NCH):
        slot = ch % 2
        if ch + 1 < _NCH:
            cps.append(start_gather(ch + 1, 1 - slot))
        for d in cps[ch]:
            d.wait()
        xs = xbufs[slot]

        for g16 in range(_CH // 16):
            def row_body(r16, carry):
                mvec, svec = carry
                ro = (g16 * 16 + r16) * _V

                def mx_body(k, acc):
                    return jnp.maximum(acc, xs[pl.ds(ro + k * 16, 16)])

                mx = lax.fori_loop(1, _NVR, mx_body, xs[pl.ds(ro, 16)])
                mx = jnp.maximum(mx, xs[pl.ds(ro + _TAIL, 16)])
                m_r = lane_max(mx)

                def sm_body(k, acc):
                    return acc + jnp.exp(xs[pl.ds(ro + k * 16, 16)] - m_r)

                sv = lax.fori_loop(0, _NVR, sm_body,
                                   jnp.zeros((16,), jnp.float32))
                tail = jnp.where(iota >= 8,
                                 jnp.exp(xs[pl.ds(ro + _TAIL, 16)] - m_r),
                                 0.0)
                s_r = lane_sum(sv + tail)
                mvec = jnp.where(iota == r16, m_r, mvec)
                svec = jnp.where(iota == r16, s_r, svec)
                return mvec, svec

            zeros = jnp.zeros((16,), jnp.float32)
            mvec, svec = lax.fori_loop(0, 16, row_body, (zeros, zeros))
            mbuf[pl.ds(ch * _CH + g16 * 16, 16)] = mvec
            sbuf[pl.ds(ch * _CH + g16 * 16, 16)] = svec

        def g_body(r2, acc):
            gv = jnp.zeros((16,), jnp.float32)
            for l in range(16):
                row_l = 2 * r2 + (l >> 3)
                lab = lbuf[pl.ds((ch * _CH + row_l) * 16 + 8 + (l & 7),
                               16)][0]
                val = xs[pl.ds(row_l * _V + lab, 16)][0]
                gv = jnp.where(iota == l, val, gv)
            gbuf[pl.ds((ch * _CH + 2 * r2) * 8, 16)] = gv
            return acc

        lax.fori_loop(0, _CH // 2, g_body, 0)

    pltpu.sync_copy(mbuf, m_hbm.at[pl.ds(base, _RPW)])
    pltpu.sync_copy(sbuf, s_hbm.at[pl.ds(base, _RPW)])
    pltpu.sync_copy(gbuf, g_hbm.at[pl.ds(base * 8, _RPW * 8)])


_sc_call = functools.partial(
    pl.kernel,
    out_type=[
        jax.ShapeDtypeStruct((_B,), jnp.float32),
        jax.ShapeDtypeStruct((_B,), jnp.float32),
        jax.ShapeDtypeStruct((_B * 8,), jnp.float32),
    ],
    mesh=plsc.VectorSubcoreMesh(core_axis_name="c", subcore_axis_name="s"),
    scratch_types=[
        pltpu.VMEM((_CH * _V + 16,), jnp.float32),
        pltpu.VMEM((_CH * _V + 16,), jnp.float32),
        pltpu.VMEM((_RPW * 16 + 16,), jnp.int32),
        pltpu.VMEM((_RPW,), jnp.float32),
        pltpu.VMEM((_RPW,), jnp.float32),
        pltpu.VMEM((_RPW * 8,), jnp.float32),
        pltpu.SemaphoreType.DMA((2,)),
    ],
)(_sc_body)


def _fin_body(m_ref, s_ref, g_ref, index_ref, diff_ref, pred_ref, acc_ref):
    lse = m_ref[:, :] + jnp.log(s_ref[:, :])
    d = lse - g_ref[:, :]
    diff_ref[:, :] = d

    col = jax.lax.broadcasted_iota(jnp.int32, (_B, 8), 1)
    mn = jnp.min(d, axis=1, keepdims=True)
    pidx = jnp.min(jnp.where(d == mn, col, 8), axis=1, keepdims=True)
    pred_ref[:, :] = pidx

    match = (index_ref[:, 0:1] - 8) == pidx
    acc_ref[0, 0] = jnp.sum(match.astype(jnp.float32)) / _B


def kernel(output, labels, index):
    m, s, g = _sc_call(output.reshape(-1), labels.reshape(-1))
    diff, pred, acc = pl.pallas_call(
        _fin_body,
        in_specs=[
            pl.BlockSpec((_B, 1), lambda: (0, 0)),
            pl.BlockSpec((_B, 1), lambda: (0, 0)),
            pl.BlockSpec((_B, 8), lambda: (0, 0)),
            pl.BlockSpec((_B, 2), lambda: (0, 0)),
        ],
        out_specs=[
            pl.BlockSpec((_B, 8), lambda: (0, 0)),
            pl.BlockSpec((_B, 1), lambda: (0, 0)),
            pl.BlockSpec((1, 1), lambda: (0, 0), memory_space=pltpu.SMEM),
        ],
        out_shape=[
            jax.ShapeDtypeStruct((_B, 8), jnp.float32),
            jax.ShapeDtypeStruct((_B, 1), jnp.int32),
            jax.ShapeDtypeStruct((1, 1), jnp.float32),
        ],
    )(m.reshape(_B, 1), s.reshape(_B, 1), g.reshape(_B, 8), index)
    return diff, pred.reshape(_B), acc[0, 0]


# speedup vs baseline: 3.1662x; 3.1662x over previous
import functools

import jax
import jax.numpy as jnp
from jax import lax
from jax.experimental import pallas as pl
from jax.experimental.pallas import tpu as pltpu
from jax.experimental.pallas import tpu_sc as plsc

_B = 4096
_V = 1000
_NC = 2
_NS = 16
_NW = _NC * _NS
_RPW = _B // _NW
_CH = 32
_NCH = _RPW // _CH
_NVR = _V // 16
_TAIL = _V - 16


def _sc_body(out_hbm, labels_hbm, m_hbm, s_hbm, g_hbm,
             xbuf0, xbuf1, lbuf, mbuf, sbuf, gbuf, gsem):
    xbufs = (xbuf0, xbuf1)
    c = lax.axis_index("c")
    s_ax = lax.axis_index("s")
    wid = s_ax * _NC + c
    base = wid * _RPW
    iota = lax.iota(jnp.int32, 16)

    gd = lax.GatherDimensionNumbers(
        offset_dims=(), collapsed_slice_dims=(0,), start_index_map=(0,))

    def xlane(v, idx):
        return lax.gather(v, idx[:, None], gd, slice_sizes=(1,),
                          mode=lax.GatherScatterMode.PROMISE_IN_BOUNDS)

    def lane_max(v):
        for k in (8, 4, 2, 1):
            v = jnp.maximum(v, xlane(v, jnp.bitwise_xor(iota, k)))
        return v

    def lane_sum(v):
        for k in (8, 4, 2, 1):
            v = v + xlane(v, jnp.bitwise_xor(iota, k))
        return v

    def start_gather(ch, slot):
        return [
            pltpu.async_copy(
                out_hbm.at[pl.ds((base + ch * _CH) * _V, _CH * _V)],
                xbufs[slot].at[pl.ds(0, _CH * _V)],
                gsem.at[slot],
            )
        ]

    cps = [start_gather(0, 0)]
    pltpu.sync_copy(labels_hbm.at[pl.ds(base * 16, _RPW * 16)],
                    lbuf.at[pl.ds(0, _RPW * 16)])

    for ch in range(_NCH):
        slot = ch % 2
        if ch + 1 < _NCH:
            cps.append(start_gather(ch + 1, 1 - slot))
        for d in cps[ch]:
            d.wait()
        xs = xbufs[slot]

        for g16 in range(_CH // 16):
            def row_body(r16, carry):
                mvec, svec = carry
                ro = (g16 * 16 + r16) * _V

                mx = xs[pl.ds(ro, 16)]
                for k in range(1, _NVR):
                    mx = jnp.maximum(mx, xs[pl.ds(ro + k * 16, 16)])
                mx = jnp.maximum(mx, xs[pl.ds(ro + _TAIL, 16)])
                m_r = lane_max(mx)

                sv = jnp.zeros((16,), jnp.float32)
                for k in range(_NVR):
                    sv = sv + jnp.exp(xs[pl.ds(ro + k * 16, 16)] - m_r)
                tail = jnp.where(iota >= 8,
                                 jnp.exp(xs[pl.ds(ro + _TAIL, 16)] - m_r),
                                 0.0)
                s_r = lane_sum(sv + tail)
                mvec = jnp.where(iota == r16, m_r, mvec)
                svec = jnp.where(iota == r16, s_r, svec)
                return mvec, svec

            zeros = jnp.zeros((16,), jnp.float32)
            mvec, svec = lax.fori_loop(0, 16, row_body, (zeros, zeros))
            mbuf[pl.ds(ch * _CH + g16 * 16, 16)] = mvec
            sbuf[pl.ds(ch * _CH + g16 * 16, 16)] = svec

        def g_body(r2, acc):
            gv = jnp.zeros((16,), jnp.float32)
            for l in range(16):
                row_l = 2 * r2 + (l >> 3)
                lab = lbuf[pl.ds((ch * _CH + row_l) * 16 + 8 + (l & 7),
                               16)][0]
                val = xs[pl.ds(row_l * _V + lab, 16)][0]
                gv = jnp.where(iota == l, val, gv)
            gbuf[pl.ds((ch * _CH + 2 * r2) * 8, 16)] = gv
            return acc

        lax.fori_loop(0, _CH // 2, g_body, 0)

    pltpu.sync_copy(mbuf, m_hbm.at[pl.ds(base, _RPW)])
    pltpu.sync_copy(sbuf, s_hbm.at[pl.ds(base, _RPW)])
    pltpu.sync_copy(gbuf, g_hbm.at[pl.ds(base * 8, _RPW * 8)])


_sc_call = functools.partial(
    pl.kernel,
    out_type=[
        jax.ShapeDtypeStruct((_B,), jnp.float32),
        jax.ShapeDtypeStruct((_B,), jnp.float32),
        jax.ShapeDtypeStruct((_B * 8,), jnp.float32),
    ],
    mesh=plsc.VectorSubcoreMesh(core_axis_name="c", subcore_axis_name="s"),
    scratch_types=[
        pltpu.VMEM((_CH * _V + 16,), jnp.float32),
        pltpu.VMEM((_CH * _V + 16,), jnp.float32),
        pltpu.VMEM((_RPW * 16 + 16,), jnp.int32),
        pltpu.VMEM((_RPW,), jnp.float32),
        pltpu.VMEM((_RPW,), jnp.float32),
        pltpu.VMEM((_RPW * 8,), jnp.float32),
        pltpu.SemaphoreType.DMA((2,)),
    ],
)(_sc_body)


def _fin_body(m_ref, s_ref, g_ref, index_ref, diff_ref, pred_ref, acc_ref):
    lse = m_ref[:, :] + jnp.log(s_ref[:, :])
    d = lse - g_ref[:, :]
    diff_ref[:, :] = d

    col = jax.lax.broadcasted_iota(jnp.int32, (_B, 8), 1)
    mn = jnp.min(d, axis=1, keepdims=True)
    pidx = jnp.min(jnp.where(d == mn, col, 8), axis=1, keepdims=True)
    pred_ref[:, :] = pidx

    match = (index_ref[:, 0:1] - 8) == pidx
    acc_ref[0, 0] = jnp.sum(match.astype(jnp.float32)) / _B


def kernel(output, labels, index):
    last_flat = output[:, 7, :].reshape(-1)
    m, s, g = _sc_call(last_flat, labels.reshape(-1))
    diff, pred, acc = pl.pallas_call(
        _fin_body,
        in_specs=[
            pl.BlockSpec((_B, 1), lambda: (0, 0)),
            pl.BlockSpec((_B, 1), lambda: (0, 0)),
            pl.BlockSpec((_B, 8), lambda: (0, 0)),
            pl.BlockSpec((_B, 2), lambda: (0, 0)),
        ],
        out_specs=[
            pl.BlockSpec((_B, 8), lambda: (0, 0)),
            pl.BlockSpec((_B, 1), lambda: (0, 0)),
            pl.BlockSpec((1, 1), lambda: (0, 0), memory_space=pltpu.SMEM),
        ],
        out_shape=[
            jax.ShapeDtypeStruct((_B, 8), jnp.float32),
            jax.ShapeDtypeStruct((_B, 1), jnp.int32),
            jax.ShapeDtypeStruct((1, 1), jnp.float32),
        ],
    )(m.reshape(_B, 1), s.reshape(_B, 1), g.reshape(_B, 8), index)
    return diff, pred.reshape(_B), acc[0, 0]


# speedup vs baseline: 3.3249x; 1.0501x over previous
import functools

import jax
import jax.numpy as jnp
from jax import lax
from jax.experimental import pallas as pl
from jax.experimental.pallas import tpu as pltpu
from jax.experimental.pallas import tpu_sc as plsc

_B = 4096
_V = 1000
_NC = 2
_NS = 16
_NW = _NC * _NS
_RPW = _B // _NW
_CH = 32
_NCH = _RPW // _CH
_NVR = _V // 16
_TAIL = _V - 16


def _sc_body(out_hbm, labels_hbm, m_hbm, s_hbm, g_hbm,
             xbuf0, xbuf1, lbuf, mbuf, sbuf, gbuf, gsem):
    xbufs = (xbuf0, xbuf1)
    c = lax.axis_index("c")
    s_ax = lax.axis_index("s")
    wid = s_ax * _NC + c
    base = wid * _RPW
    iota = lax.iota(jnp.int32, 16)

    gd = lax.GatherDimensionNumbers(
        offset_dims=(), collapsed_slice_dims=(0,), start_index_map=(0,))

    def xlane(v, idx):
        return lax.gather(v, idx[:, None], gd, slice_sizes=(1,),
                          mode=lax.GatherScatterMode.PROMISE_IN_BOUNDS)

    def lane_max(v):
        for k in (8, 4, 2, 1):
            v = jnp.maximum(v, xlane(v, jnp.bitwise_xor(iota, k)))
        return v

    def lane_sum(v):
        for k in (8, 4, 2, 1):
            v = v + xlane(v, jnp.bitwise_xor(iota, k))
        return v

    def start_gather(ch, slot):
        return [
            pltpu.async_copy(
                out_hbm.at[pl.ds((base + ch * _CH) * _V, _CH * _V)],
                xbufs[slot].at[pl.ds(0, _CH * _V)],
                gsem.at[slot],
            )
        ]

    cps = [start_gather(0, 0)]
    pltpu.sync_copy(labels_hbm.at[pl.ds(base * 16, _RPW * 16)],
                    lbuf.at[pl.ds(0, _RPW * 16)])

    for ch in range(_NCH):
        slot = ch % 2
        if ch + 1 < _NCH:
            cps.append(start_gather(ch + 1, 1 - slot))
        for d in cps[ch]:
            d.wait()
        xs = xbufs[slot]

        for g16 in range(_CH // 16):
            def row_body(r16, carry):
                mvec, svec = carry
                ro = (g16 * 16 + r16) * _V

                mxa = [xs[pl.ds(ro + k * 16, 16)] for k in range(4)]
                for k in range(4, _NVR):
                    mxa[k % 4] = jnp.maximum(mxa[k % 4],
                                             xs[pl.ds(ro + k * 16, 16)])
                mx = jnp.maximum(jnp.maximum(mxa[0], mxa[1]),
                                 jnp.maximum(mxa[2], mxa[3]))
                mx = jnp.maximum(mx, xs[pl.ds(ro + _TAIL, 16)])
                m_r = lane_max(mx)

                sva = [jnp.zeros((16,), jnp.float32) for _ in range(4)]
                for k in range(_NVR):
                    sva[k % 4] = sva[k % 4] + jnp.exp(
                        xs[pl.ds(ro + k * 16, 16)] - m_r)
                sv = (sva[0] + sva[1]) + (sva[2] + sva[3])
                tail = jnp.where(iota >= 8,
                                 jnp.exp(xs[pl.ds(ro + _TAIL, 16)] - m_r),
                                 0.0)
                s_r = lane_sum(sv + tail)
                mvec = jnp.where(iota == r16, m_r, mvec)
                svec = jnp.where(iota == r16, s_r, svec)
                return mvec, svec

            zeros = jnp.zeros((16,), jnp.float32)
            mvec, svec = lax.fori_loop(0, 16, row_body, (zeros, zeros))
            mbuf[pl.ds(ch * _CH + g16 * 16, 16)] = mvec
            sbuf[pl.ds(ch * _CH + g16 * 16, 16)] = svec

        def g_body(r2, acc):
            gv = jnp.zeros((16,), jnp.float32)
            for l in range(16):
                row_l = 2 * r2 + (l >> 3)
                lab = lbuf[pl.ds((ch * _CH + row_l) * 16 + 8 + (l & 7),
                               16)][0]
                val = xs[pl.ds(row_l * _V + lab, 16)][0]
                gv = jnp.where(iota == l, val, gv)
            gbuf[pl.ds((ch * _CH + 2 * r2) * 8, 16)] = gv
            return acc

        lax.fori_loop(0, _CH // 2, g_body, 0)

    pltpu.sync_copy(mbuf, m_hbm.at[pl.ds(base, _RPW)])
    pltpu.sync_copy(sbuf, s_hbm.at[pl.ds(base, _RPW)])
    pltpu.sync_copy(gbuf, g_hbm.at[pl.ds(base * 8, _RPW * 8)])


_sc_call = functools.partial(
    pl.kernel,
    out_type=[
        jax.ShapeDtypeStruct((_B,), jnp.float32),
        jax.ShapeDtypeStruct((_B,), jnp.float32),
        jax.ShapeDtypeStruct((_B * 8,), jnp.float32),
    ],
    mesh=plsc.VectorSubcoreMesh(core_axis_name="c", subcore_axis_name="s"),
    scratch_types=[
        pltpu.VMEM((_CH * _V + 16,), jnp.float32),
        pltpu.VMEM((_CH * _V + 16,), jnp.float32),
        pltpu.VMEM((_RPW * 16 + 16,), jnp.int32),
        pltpu.VMEM((_RPW,), jnp.float32),
        pltpu.VMEM((_RPW,), jnp.float32),
        pltpu.VMEM((_RPW * 8,), jnp.float32),
        pltpu.SemaphoreType.DMA((2,)),
    ],
)(_sc_body)


def _fin_body(m_ref, s_ref, g_ref, index_ref, diff_ref, pred_ref, acc_ref):
    lse = m_ref[:, :] + jnp.log(s_ref[:, :])
    d = lse - g_ref[:, :]
    diff_ref[:, :] = d

    col = jax.lax.broadcasted_iota(jnp.int32, (_B, 8), 1)
    mn = jnp.min(d, axis=1, keepdims=True)
    pidx = jnp.min(jnp.where(d == mn, col, 8), axis=1, keepdims=True)
    pred_ref[:, :] = pidx

    match = (index_ref[:, 0:1] - 8) == pidx
    acc_ref[0, 0] = jnp.sum(match.astype(jnp.float32)) / _B


def kernel(output, labels, index):
    last_flat = output[:, 7, :].reshape(-1)
    m, s, g = _sc_call(last_flat, labels.reshape(-1))
    diff, pred, acc = pl.pallas_call(
        _fin_body,
        in_specs=[
            pl.BlockSpec((_B, 1), lambda: (0, 0)),
            pl.BlockSpec((_B, 1), lambda: (0, 0)),
            pl.BlockSpec((_B, 8), lambda: (0, 0)),
            pl.BlockSpec((_B, 2), lambda: (0, 0)),
        ],
        out_specs=[
            pl.BlockSpec((_B, 8), lambda: (0, 0)),
            pl.BlockSpec((_B, 1), lambda: (0, 0)),
            pl.BlockSpec((1, 1), lambda: (0, 0), memory_space=pltpu.SMEM),
        ],
        out_shape=[
            jax.ShapeDtypeStruct((_B, 8), jnp.float32),
            jax.ShapeDtypeStruct((_B, 1), jnp.int32),
            jax.ShapeDtypeStruct((1, 1), jnp.float32),
        ],
    )(m.reshape(_B, 1), s.reshape(_B, 1), g.reshape(_B, 8), index)
    return diff, pred.reshape(_B), acc[0, 0]


# speedup vs baseline: 3.8312x; 1.1523x over previous
import functools

import jax
import jax.numpy as jnp
from jax import lax
from jax.experimental import pallas as pl
from jax.experimental.pallas import tpu as pltpu
from jax.experimental.pallas import tpu_sc as plsc

_B = 4096
_V = 1000
_NC = 2
_NS = 16
_NW = _NC * _NS
_RPW = _B // _NW
_CH = 32
_NCH = _RPW // _CH
_NVR = _V // 16
_TAIL = _V - 16


def _sc_body(out_hbm, labels_hbm, m_hbm, s_hbm, g_hbm,
             xbuf0, xbuf1, lbuf, mbuf, sbuf, gbuf, gsem):
    xbufs = (xbuf0, xbuf1)
    c = lax.axis_index("c")
    s_ax = lax.axis_index("s")
    wid = s_ax * _NC + c
    base = wid * _RPW
    iota = lax.iota(jnp.int32, 16)

    gd = lax.GatherDimensionNumbers(
        offset_dims=(), collapsed_slice_dims=(0,), start_index_map=(0,))

    def xlane(v, idx):
        return lax.gather(v, idx[:, None], gd, slice_sizes=(1,),
                          mode=lax.GatherScatterMode.PROMISE_IN_BOUNDS)

    def lane_max(v):
        for k in (8, 4, 2, 1):
            v = jnp.maximum(v, xlane(v, jnp.bitwise_xor(iota, k)))
        return v

    def lane_sum(v):
        for k in (8, 4, 2, 1):
            v = v + xlane(v, jnp.bitwise_xor(iota, k))
        return v

    def start_gather(ch, slot):
        return [
            pltpu.async_copy(
                out_hbm.at[pl.ds((base + ch * _CH) * _V, _CH * _V)],
                xbufs[slot].at[pl.ds(0, _CH * _V)],
                gsem.at[slot],
            )
        ]

    cps = [start_gather(0, 0)]
    pltpu.sync_copy(labels_hbm.at[pl.ds(base * 16, _RPW * 16)],
                    lbuf.at[pl.ds(0, _RPW * 16)])

    for ch in range(_NCH):
        slot = ch % 2
        if ch + 1 < _NCH:
            cps.append(start_gather(ch + 1, 1 - slot))
        for d in cps[ch]:
            d.wait()
        xs = xbufs[slot]

        for g16 in range(_CH // 16):
            def row_body(r16, carry):
                mvec, svec = carry
                ro = (g16 * 16 + r16) * _V

                mxa = [xs[pl.ds(ro + k * 16, 16)] for k in range(4)]
                for k in range(4, _NVR):
                    mxa[k % 4] = jnp.maximum(mxa[k % 4],
                                             xs[pl.ds(ro + k * 16, 16)])
                mx = jnp.maximum(jnp.maximum(mxa[0], mxa[1]),
                                 jnp.maximum(mxa[2], mxa[3]))
                mx = jnp.maximum(mx, xs[pl.ds(ro + _TAIL, 16)])
                m_r = lane_max(mx)

                sva = [jnp.zeros((16,), jnp.float32) for _ in range(4)]
                for k in range(_NVR):
                    sva[k % 4] = sva[k % 4] + jnp.exp(
                        xs[pl.ds(ro + k * 16, 16)] - m_r)
                sv = (sva[0] + sva[1]) + (sva[2] + sva[3])
                tail = jnp.where(iota >= 8,
                                 jnp.exp(xs[pl.ds(ro + _TAIL, 16)] - m_r),
                                 0.0)
                s_r = lane_sum(sv + tail)
                mvec = jnp.where(iota == r16, m_r, mvec)
                svec = jnp.where(iota == r16, s_r, svec)
                return mvec, svec

            zeros = jnp.zeros((16,), jnp.float32)
            mvec, svec = lax.fori_loop(0, 16, row_body, (zeros, zeros))
            mbuf[pl.ds(ch * _CH + g16 * 16, 16)] = mvec
            sbuf[pl.ds(ch * _CH + g16 * 16, 16)] = svec

        def g_body(it, acc):
            i = jnp.bitwise_and(it, 7)
            grp = jnp.right_shift(it, 3)
            gv = jnp.zeros((16,), jnp.float32)
            for l in range(16):
                row_l = grp * 16 + l
                lab = lbuf[pl.ds((ch * _CH + row_l) * 16 + 8 + i, 16)][0]
                val = xs[pl.ds(row_l * _V + lab, 16)][0]
                gv = jnp.where(iota == l, val, gv)
            gbuf[pl.ds(i * _RPW + ch * _CH + grp * 16, 16)] = gv
            return acc

        lax.fori_loop(0, (_CH // 16) * 8, g_body, 0)

    pltpu.sync_copy(mbuf, m_hbm.at[pl.ds(base, _RPW)])
    pltpu.sync_copy(sbuf, s_hbm.at[pl.ds(base, _RPW)])
    for i in range(8):
        pltpu.sync_copy(gbuf.at[pl.ds(i * _RPW, _RPW)],
                        g_hbm.at[pl.ds(i * _B + base, _RPW)])


_sc_call = functools.partial(
    pl.kernel,
    out_type=[
        jax.ShapeDtypeStruct((_B,), jnp.float32),
        jax.ShapeDtypeStruct((_B,), jnp.float32),
        jax.ShapeDtypeStruct((_B * 8,), jnp.float32),
    ],
    mesh=plsc.VectorSubcoreMesh(core_axis_name="c", subcore_axis_name="s"),
    scratch_types=[
        pltpu.VMEM((_CH * _V + 16,), jnp.float32),
        pltpu.VMEM((_CH * _V + 16,), jnp.float32),
        pltpu.VMEM((_RPW * 16 + 16,), jnp.int32),
        pltpu.VMEM((_RPW,), jnp.float32),
        pltpu.VMEM((_RPW,), jnp.float32),
        pltpu.VMEM((_RPW * 8,), jnp.float32),
        pltpu.SemaphoreType.DMA((2,)),
    ],
)(_sc_body)


_R = _B // 128


def _fin_body(m_ref, s_ref, g_ref, idx0_ref, diff_ref, pred_ref, acc_ref):
    lse = m_ref[:, :] + jnp.log(s_ref[:, :])
    mn = lse - g_ref[0, :, :]
    diff_ref[0, :, :] = mn
    pidx = jnp.zeros((_R, 128), jnp.int32)
    for i in range(1, 8):
        d_i = lse - g_ref[i, :, :]
        diff_ref[i, :, :] = d_i
        lt = d_i < mn
        mn = jnp.where(lt, d_i, mn)
        pidx = jnp.where(lt, i, pidx)
    pred_ref[:, :] = pidx

    match = (idx0_ref[:, :] - 8) == pidx
    acc_ref[0, 0] = jnp.sum(match.astype(jnp.float32)) / _B


def kernel(output, labels, index):
    last_flat = output[:, 7, :].reshape(-1)
    m, s, g = _sc_call(last_flat, labels.reshape(-1))
    diff_t, pred, acc = pl.pallas_call(
        _fin_body,
        in_specs=[
            pl.BlockSpec((_R, 128), lambda: (0, 0)),
            pl.BlockSpec((_R, 128), lambda: (0, 0)),
            pl.BlockSpec((8, _R, 128), lambda: (0, 0, 0)),
            pl.BlockSpec((_R, 128), lambda: (0, 0)),
        ],
        out_specs=[
            pl.BlockSpec((8, _R, 128), lambda: (0, 0, 0)),
            pl.BlockSpec((_R, 128), lambda: (0, 0)),
            pl.BlockSpec((1, 1), lambda: (0, 0), memory_space=pltpu.SMEM),
        ],
        out_shape=[
            jax.ShapeDtypeStruct((8, _R, 128), jnp.float32),
            jax.ShapeDtypeStruct((_R, 128), jnp.int32),
            jax.ShapeDtypeStruct((1, 1), jnp.float32),
        ],
    )(m.reshape(_R, 128), s.reshape(_R, 128), g.reshape(8, _R, 128),
      index[:, 0].reshape(_R, 128))
    diff = jnp.swapaxes(diff_t.reshape(8, _B), 0, 1)
    return diff, pred.reshape(_B), acc[0, 0]
